# baseline (speedup 1.0000x reference)
import functools

import jax
import jax.numpy as jnp
from jax import lax
from jax.experimental import pallas as pl
from jax.experimental.pallas import tpu as pltpu
from jax.experimental.pallas import tpu_sc as plsc

NV = 10000
NF = 10000
E = 320000
D = 128
G = 16

NCORE = 2
NSUB = 16
NW = NCORE * NSUB
EPW = E // NW
K = 80
NCHUNK = EPW // K
NPAIR = (NCHUNK - 1) // 2
NRC = NF // K
NRC_PT = -(-NRC // NSUB)

_SC_MESH = plsc.VectorSubcoreMesh(
    core_axis_name="c", subcore_axis_name="s",
    num_cores=NCORE, num_subcores=NSUB)


def _compute_m(rows_i, rows_j):
    def _body(e2, _):
        e = e2 * 2
        for ee in (0, 1):
            for cc in range(D // 16):
                sl = pl.ds(cc * 16, 16)
                rows_i[e + ee, sl] = jnp.maximum(
                    rows_i[e + ee, sl] + rows_j[e + ee, sl], 0.0)
        return 0
    lax.fori_loop(0, K // 2, _body, 0)


def _seg_stage(ti_hbm, tj_hbm, idxi_hbm, idxj_hbm, out_hbm,
               acc, xbufs, rows, xsems, rsems, ssems, c, s, w):
    (xi0, xj0), (xi1, xj1) = xbufs
    (ri0, rj0), (ri1, rj1) = rows
    (sxi0, sxj0), (sxi1, sxj1) = xsems
    (si0, sj0), (si1, sj1) = rsems
    ss0, ss1 = ssems

    def _zero(e, _):
        for cc in range(D // 16):
            ri0[e, pl.ds(cc * 16, 16)] = jnp.zeros((16,), jnp.float32)
        return 0
    lax.fori_loop(0, K, _zero, 0)
    for j in range(NRC_PT):
        cid = s + j * NSUB

        @pl.when(cid < NRC)
        def _():
            pltpu.sync_copy(ri0, acc.at[pl.ds(cid * K, K)])
    plsc.subcore_barrier()

    def _idxload(t, xi, xj, sxi, sxj):
        pltpu.async_copy(idxi_hbm.at[w, t], xi, sxi)
        pltpu.async_copy(idxj_hbm.at[w, t], xj, sxj)

    def _wait_idx(t, xi, xj, sxi, sxj):
        pltpu.make_async_copy(idxi_hbm.at[w, t], xi, sxi).wait()
        pltpu.make_async_copy(idxj_hbm.at[w, t], xj, sxj).wait()

    def _gather(xi, xj, ri, rj, semi, semj):
        pltpu.async_copy(ti_hbm.at[xi], ri, semi)
        pltpu.async_copy(tj_hbm.at[xj], rj, semj)

    def _consume(xi, xj, ri, rj, semi, semj, ss):
        pltpu.make_async_copy(ti_hbm.at[xi], ri, semi).wait()
        pltpu.make_async_copy(tj_hbm.at[xj], rj, semj).wait()
        _compute_m(ri, rj)
        pltpu.async_copy(ri, acc.at[xi], ss)

    def _drain(xi, ri, ss):
        pltpu.make_async_copy(ri, acc.at[xi], ss).wait()

    _idxload(0, xi0, xj0, sxi0, sxj0)
    _wait_idx(0, xi0, xj0, sxi0, sxj0)
    _gather(xi0, xj0, ri0, rj0, si0, sj0)
    _idxload(1, xi1, xj1, sxi1, sxj1)

    def _pair_body(t, _):
        c1 = 2 * t + 1
        _wait_idx(c1, xi1, xj1, sxi1, sxj1)
        _gather(xi1, xj1, ri1, rj1, si1, sj1)
        _consume(xi0, xj0, ri0, rj0, si0, sj0, ss0)
        _consume(xi1, xj1, ri1, rj1, si1, sj1, ss1)
        _drain(xi0, ri0, ss0)
        _idxload(c1 + 1, xi0, xj0, sxi0, sxj0)
        _wait_idx(c1 + 1, xi0, xj0, sxi0, sxj0)
        _gather(xi0, xj0, ri0, rj0, si0, sj0)
        _drain(xi1, ri1, ss1)

        @pl.when(t + 1 < NPAIR)
        def _():
            _idxload(c1 + 2, xi1, xj1, sxi1, sxj1)
        return 0
    lax.fori_loop(0, NPAIR, _pair_body, 0)
    _consume(xi0, xj0, ri0, rj0, si0, sj0, ss0)
    _drain(xi0, ri0, ss0)
    plsc.subcore_barrier()

    for j in range(NRC_PT):
        cid = s + j * NSUB

        @pl.when(cid < NRC)
        def _():
            pltpu.sync_copy(acc.at[pl.ds(cid * K, K)],
                            out_hbm.at[c, pl.ds(cid * K, K)])


def _edges_body(a_hbm, b_hbm, c_hbm, d_hbm, faci_hbm, vari_hbm,
                aggf_hbm, aggv_hbm,
                acc, xi0, xj0, xi1, xj1, ri0, rj0, ri1, rj1,
                sxi0, sxj0, sxi1, sxj1, si0, sj0, si1, sj1, ss0, ss1):
    c = lax.axis_index("c")
    s = lax.axis_index("s")
    w = c * NSUB + s
    xbufs = ((xi0, xj0), (xi1, xj1))
    rows = ((ri0, rj0), (ri1, rj1))
    xsems = ((sxi0, sxj0), (sxi1, sxj1))
    rsems = ((si0, sj0), (si1, sj1))
    ssems = (ss0, ss1)
    _seg_stage(a_hbm, b_hbm, faci_hbm, vari_hbm, aggf_hbm,
               acc, xbufs, rows, xsems, rsems, ssems, c, s, w)
    plsc.subcore_barrier()
    _seg_stage(c_hbm, d_hbm, vari_hbm, faci_hbm, aggv_hbm,
               acc, xbufs, rows, xsems, rsems, ssems, c, s, w)


_edges_call = functools.partial(
    pl.kernel,
    out_type=(jax.ShapeDtypeStruct((NCORE, NF, D), jnp.float32),
              jax.ShapeDtypeStruct((NCORE, NV, D), jnp.float32)),
    mesh=_SC_MESH,
    scratch_types=[
        pltpu.VMEM_SHARED((NF, D), jnp.float32),
        pltpu.VMEM((K,), jnp.int32),
        pltpu.VMEM((K,), jnp.int32),
        pltpu.VMEM((K,), jnp.int32),
        pltpu.VMEM((K,), jnp.int32),
        pltpu.VMEM((K, D), jnp.float32),
        pltpu.VMEM((K, D), jnp.float32),
        pltpu.VMEM((K, D), jnp.float32),
        pltpu.VMEM((K, D), jnp.float32),
        pltpu.SemaphoreType.DMA,
        pltpu.SemaphoreType.DMA,
        pltpu.SemaphoreType.DMA,
        pltpu.SemaphoreType.DMA,
        pltpu.SemaphoreType.DMA,
        pltpu.SemaphoreType.DMA,
        pltpu.SemaphoreType.DMA,
        pltpu.SemaphoreType.DMA,
        pltpu.SemaphoreType.DMA,
        pltpu.SemaphoreType.DMA,
    ],
)(_edges_body)


_BS = 1000
_GRID = NF // _BS


def _pre_body(fac_ref, var_ref, wia_ref, ba_ref, wjb_ref, wic_ref, bc_ref,
              wjd_ref, a_ref, b_ref, c_ref, d_ref):
    f = fac_ref[...]
    v = var_ref[...]
    a_ref[...] = jnp.dot(f, wia_ref[...], preferred_element_type=jnp.float32) + ba_ref[...]
    b_ref[...] = jnp.dot(v, wjb_ref[...], preferred_element_type=jnp.float32)
    c_ref[...] = jnp.dot(v, wic_ref[...], preferred_element_type=jnp.float32) + bc_ref[...]
    d_ref[...] = jnp.dot(f, wjd_ref[...], preferred_element_type=jnp.float32)


def _comb_body(fac_ref, var_ref, aggf_ref, aggv_ref,
               wf1_ref, wf2_ref, bf_ref, wv1_ref, wv2_ref, bv_ref,
               nf_ref, nv_ref):
    f = fac_ref[...]
    v = var_ref[...]
    aggf = aggf_ref[0] + aggf_ref[1]
    aggv = aggv_ref[0] + aggv_ref[1]
    nf = jnp.dot(f, wf1_ref[...], preferred_element_type=jnp.float32)
    nf = nf + jnp.dot(aggf, wf2_ref[...], preferred_element_type=jnp.float32) + bf_ref[...]
    nf_ref[...] = jnp.maximum(nf, 0.0)
    nv = jnp.dot(v, wv1_ref[...], preferred_element_type=jnp.float32)
    nv = nv + jnp.dot(aggv, wv2_ref[...], preferred_element_type=jnp.float32) + bv_ref[...]
    nv_ref[...] = v + jnp.maximum(nv, 0.0)


def _global_body(fac_ref, batch_ref, gw_ref, gb_ref, nnw_ref, nnb_ref,
                 lw_ref, lb_ref, g_ref):
    f = fac_ref[...]
    gate = jnp.dot(f, gw_ref[...], preferred_element_type=jnp.float32) + gb_ref[...]
    mask = batch_ref[...] == lax.broadcasted_iota(jnp.int32, (NF, G), 1)
    gmax = jnp.max(jnp.where(mask, gate, jnp.float32(-1e30)), axis=0, keepdims=True)
    gmax_f = jnp.sum(jnp.where(mask, gmax, 0.0), axis=1, keepdims=True)
    ex = jnp.exp(gate - gmax_f)
    denom = jnp.sum(jnp.where(mask, ex, 0.0), axis=0, keepdims=True)
    denom_f = jnp.sum(jnp.where(mask, denom, 0.0), axis=1, keepdims=True)
    alpha = ex / denom_f
    val = jnp.dot(f, nnw_ref[...], preferred_element_type=jnp.float32) + nnb_ref[...]
    wmat = jnp.where(mask, alpha, 0.0)
    g_agg = lax.dot_general(wmat, val, (((0,), (0,)), ((), ())),
                            preferred_element_type=jnp.float32)
    g = jnp.dot(g_agg, lw_ref[...], preferred_element_type=jnp.float32) + lb_ref[...]
    g_ref[...] = jnp.maximum(g, 0.0)


def _row_spec():
    return pl.BlockSpec((_BS, D), lambda i: (i, 0))


_W_SPEC = pl.BlockSpec((D, D), lambda i: (0, 0))
_B_SPEC = pl.BlockSpec((1, D), lambda i: (0, 0))
_AGG_SPEC = pl.BlockSpec((NCORE, _BS, D), lambda i: (0, i, 0))


def _pre_call(factors, variables, wia, ba, wjb, wic, bc, wjd):
    return pl.pallas_call(
        _pre_body,
        grid=(_GRID,),
        in_specs=[_row_spec(), _row_spec(), _W_SPEC, _B_SPEC, _W_SPEC,
                  _W_SPEC, _B_SPEC, _W_SPEC],
        out_specs=[_row_spec(), _row_spec(), _row_spec(), _row_spec()],
        out_shape=[jax.ShapeDtypeStruct((NF, D), jnp.float32),
                   jax.ShapeDtypeStruct((NV, D), jnp.float32),
                   jax.ShapeDtypeStruct((NV, D), jnp.float32),
                   jax.ShapeDtypeStruct((NF, D), jnp.float32)],
    )(factors, variables, wia, ba, wjb, wic, bc, wjd)


def _comb_call(factors, variables, aggfP, aggvP, wf1, wf2, bf, wv1, wv2, bv):
    return pl.pallas_call(
        _comb_body,
        grid=(_GRID,),
        in_specs=[_row_spec(), _row_spec(), _AGG_SPEC, _AGG_SPEC,
                  _W_SPEC, _W_SPEC, _B_SPEC, _W_SPEC, _W_SPEC, _B_SPEC],
        out_specs=[_row_spec(), _row_spec()],
        out_shape=[jax.ShapeDtypeStruct((NF, D), jnp.float32),
                   jax.ShapeDtypeStruct((NV, D), jnp.float32)],
    )(factors, variables, aggfP, aggvP, wf1, wf2, bf, wv1, wv2, bv)


def _global_call(factors, batch2d, gw, gb, nnw, nnb, lw, lb):
    return pl.pallas_call(
        _global_body,
        in_specs=[pl.BlockSpec((NF, D), lambda: (0, 0)),
                  pl.BlockSpec((NF, 1), lambda: (0, 0)),
                  pl.BlockSpec((D, 1), lambda: (0, 0)),
                  pl.BlockSpec((1, 1), lambda: (0, 0)),
                  pl.BlockSpec((D, D), lambda: (0, 0)),
                  pl.BlockSpec((1, D), lambda: (0, 0)),
                  pl.BlockSpec((D, D), lambda: (0, 0)),
                  pl.BlockSpec((1, D), lambda: (0, 0))],
        out_specs=pl.BlockSpec((G, D), lambda: (0, 0)),
        out_shape=jax.ShapeDtypeStruct((G, D), jnp.float32),
    )(factors, batch2d, gw, gb, nnw, nnb, lw, lb)


def kernel(variables, factors, edge_index, edge_attr, batch_idx,
           v2f_msg_W, v2f_msg_b, v2f_comb_W, v2f_comb_b,
           f2v_msg_W, f2v_msg_b, f2v_comb_W, f2v_comb_b,
           gate_W, gate_b, nn_W, nn_b, lin_W, lin_b):
    del edge_attr
    fac_idx = edge_index[1].astype(jnp.int32).reshape(NW, NCHUNK, K)
    var_idx = edge_index[0].astype(jnp.int32).reshape(NW, NCHUNK, K)
    batch2d = batch_idx.astype(jnp.int32).reshape(NF, 1)

    for l in range(2):
        a_tab, b_tab, c_tab, d_tab = _pre_call(
            factors, variables,
            v2f_msg_W[l, :D], v2f_msg_b[l].reshape(1, D), v2f_msg_W[l, D:],
            f2v_msg_W[l, :D], f2v_msg_b[l].reshape(1, D), f2v_msg_W[l, D:])
        aggf, aggv = _edges_call(a_tab, b_tab, c_tab, d_tab, fac_idx, var_idx)
        factors, variables = _comb_call(
            factors, variables, aggf, aggv,
            v2f_comb_W[l, :D], v2f_comb_W[l, D:], v2f_comb_b[l].reshape(1, D),
            f2v_comb_W[l, :D], f2v_comb_W[l, D:], f2v_comb_b[l].reshape(1, D))

    g = _global_call(factors, batch2d, gate_W, gate_b.reshape(1, 1),
                     nn_W, nn_b.reshape(1, D), lin_W[:D], lin_b.reshape(1, D))
    return (variables, factors, g)


# speedup vs baseline: 1.1138x; 1.1138x over previous
import functools

import jax
import jax.numpy as jnp
from jax import lax
from jax.experimental import pallas as pl
from jax.experimental.pallas import tpu as pltpu
from jax.experimental.pallas import tpu_sc as plsc

NV = 10000
NF = 10000
E = 320000
D = 128
G = 16

NCORE = 2
NSUB = 16
NW = NCORE * NSUB
EPW = E // NW
K = 80
NCHUNK = EPW // K
NPAIR = (NCHUNK - 1) // 2
NRC = NF // K
NRC_PT = -(-NRC // NSUB)

_SC_MESH = plsc.VectorSubcoreMesh(
    core_axis_name="c", subcore_axis_name="s",
    num_cores=NCORE, num_subcores=NSUB)


def _compute_m(rows_i, rows_j):
    def _body(e, _):
        for cc in range(D // 16):
            sl = pl.ds(cc * 16, 16)
            rows_i[e, sl] = jnp.maximum(rows_i[e, sl] + rows_j[e, sl], 0.0)
        return 0
    lax.fori_loop(0, K, _body, 0)


def _seg_stage(ti_hbm, tj_hbm, idxi_hbm, idxj_hbm, out_hbm,
               acc, xbufs, rows, xsems, rsems, c, s, w):
    (xi0, xj0), (xi1, xj1) = xbufs
    (ri0, rj0), (ri1, rj1) = rows
    (sxi0, sxj0), (sxi1, sxj1) = xsems
    (si0, sj0), (si1, sj1) = rsems

    def _zero(e, _):
        for cc in range(D // 16):
            ri0[e, pl.ds(cc * 16, 16)] = jnp.zeros((16,), jnp.float32)
        return 0
    lax.fori_loop(0, K, _zero, 0)
    for j in range(NRC_PT):
        cid = s + j * NSUB

        @pl.when(cid < NRC)
        def _():
            pltpu.sync_copy(ri0, acc.at[pl.ds(cid * K, K)])
    plsc.subcore_barrier()

    def _idxload(t, xi, xj, sxi, sxj):
        pltpu.async_copy(idxi_hbm.at[w, t], xi, sxi)
        pltpu.async_copy(idxj_hbm.at[w, t], xj, sxj)

    def _wait_idx(t, xi, xj, sxi, sxj):
        pltpu.make_async_copy(idxi_hbm.at[w, t], xi, sxi).wait()
        pltpu.make_async_copy(idxj_hbm.at[w, t], xj, sxj).wait()

    def _gather(xi, xj, ri, rj, semi, semj):
        pltpu.async_copy(ti_hbm.at[xi], ri, semi)
        pltpu.async_copy(tj_hbm.at[xj], rj, semj)

    def _consume(xi, xj, ri, rj, semi, semj):
        pltpu.make_async_copy(ti_hbm.at[xi], ri, semi).wait()
        pltpu.make_async_copy(tj_hbm.at[xj], rj, semj).wait()
        _compute_m(ri, rj)
        pltpu.sync_copy(ri, acc.at[xi], add=True)

    _idxload(0, xi0, xj0, sxi0, sxj0)
    _wait_idx(0, xi0, xj0, sxi0, sxj0)
    _gather(xi0, xj0, ri0, rj0, si0, sj0)
    _idxload(1, xi1, xj1, sxi1, sxj1)

    def _pair_body(t, _):
        c1 = 2 * t + 1
        _wait_idx(c1, xi1, xj1, sxi1, sxj1)
        _gather(xi1, xj1, ri1, rj1, si1, sj1)
        _consume(xi0, xj0, ri0, rj0, si0, sj0)
        _idxload(c1 + 1, xi0, xj0, sxi0, sxj0)
        _wait_idx(c1 + 1, xi0, xj0, sxi0, sxj0)
        _gather(xi0, xj0, ri0, rj0, si0, sj0)
        _consume(xi1, xj1, ri1, rj1, si1, sj1)

        @pl.when(t + 1 < NPAIR)
        def _():
            _idxload(c1 + 2, xi1, xj1, sxi1, sxj1)
        return 0
    lax.fori_loop(0, NPAIR, _pair_body, 0)
    _consume(xi0, xj0, ri0, rj0, si0, sj0)
    plsc.subcore_barrier()

    for j in range(NRC_PT):
        cid = s + j * NSUB

        @pl.when(cid < NRC)
        def _():
            pltpu.sync_copy(acc.at[pl.ds(cid * K, K)],
                            out_hbm.at[c, pl.ds(cid * K, K)])


def _edges_body(a_hbm, b_hbm, c_hbm, d_hbm, faci_hbm, vari_hbm,
                aggf_hbm, aggv_hbm,
                acc, xi0, xj0, xi1, xj1, ri0, rj0, ri1, rj1,
                sxi0, sxj0, sxi1, sxj1, si0, sj0, si1, sj1):
    c = lax.axis_index("c")
    s = lax.axis_index("s")
    w = c * NSUB + s
    xbufs = ((xi0, xj0), (xi1, xj1))
    rows = ((ri0, rj0), (ri1, rj1))
    xsems = ((sxi0, sxj0), (sxi1, sxj1))
    rsems = ((si0, sj0), (si1, sj1))
    _seg_stage(a_hbm, b_hbm, faci_hbm, vari_hbm, aggf_hbm,
               acc, xbufs, rows, xsems, rsems, c, s, w)
    plsc.subcore_barrier()
    _seg_stage(c_hbm, d_hbm, vari_hbm, faci_hbm, aggv_hbm,
               acc, xbufs, rows, xsems, rsems, c, s, w)


_edges_call = functools.partial(
    pl.kernel,
    out_type=(jax.ShapeDtypeStruct((NCORE, NF, D), jnp.float32),
              jax.ShapeDtypeStruct((NCORE, NV, D), jnp.float32)),
    mesh=_SC_MESH,
    scratch_types=[
        pltpu.VMEM_SHARED((NF, D), jnp.float32),
        pltpu.VMEM((K,), jnp.int32),
        pltpu.VMEM((K,), jnp.int32),
        pltpu.VMEM((K,), jnp.int32),
        pltpu.VMEM((K,), jnp.int32),
        pltpu.VMEM((K, D), jnp.float32),
        pltpu.VMEM((K, D), jnp.float32),
        pltpu.VMEM((K, D), jnp.float32),
        pltpu.VMEM((K, D), jnp.float32),
        pltpu.SemaphoreType.DMA,
        pltpu.SemaphoreType.DMA,
        pltpu.SemaphoreType.DMA,
        pltpu.SemaphoreType.DMA,
        pltpu.SemaphoreType.DMA,
        pltpu.SemaphoreType.DMA,
        pltpu.SemaphoreType.DMA,
        pltpu.SemaphoreType.DMA,
    ],
)(_edges_body)


_BS = 1000
_GRID = NF // _BS


def _pre_body(fac_ref, var_ref, wia_ref, ba_ref, wjb_ref, wic_ref, bc_ref,
              wjd_ref, a_ref, b_ref, c_ref, d_ref):
    f = fac_ref[...]
    v = var_ref[...]
    a_ref[...] = jnp.dot(f, wia_ref[...], preferred_element_type=jnp.float32) + ba_ref[...]
    b_ref[...] = jnp.dot(v, wjb_ref[...], preferred_element_type=jnp.float32)
    c_ref[...] = jnp.dot(v, wic_ref[...], preferred_element_type=jnp.float32) + bc_ref[...]
    d_ref[...] = jnp.dot(f, wjd_ref[...], preferred_element_type=jnp.float32)


def _comb_body(fac_ref, var_ref, aggf_ref, aggv_ref,
               wf1_ref, wf2_ref, bf_ref, wv1_ref, wv2_ref, bv_ref,
               nf_ref, nv_ref):
    f = fac_ref[...]
    v = var_ref[...]
    aggf = aggf_ref[0] + aggf_ref[1]
    aggv = aggv_ref[0] + aggv_ref[1]
    nf = jnp.dot(f, wf1_ref[...], preferred_element_type=jnp.float32)
    nf = nf + jnp.dot(aggf, wf2_ref[...], preferred_element_type=jnp.float32) + bf_ref[...]
    nf_ref[...] = jnp.maximum(nf, 0.0)
    nv = jnp.dot(v, wv1_ref[...], preferred_element_type=jnp.float32)
    nv = nv + jnp.dot(aggv, wv2_ref[...], preferred_element_type=jnp.float32) + bv_ref[...]
    nv_ref[...] = v + jnp.maximum(nv, 0.0)


def _global_body(fac_ref, batch_ref, gw_ref, gb_ref, nnw_ref, nnb_ref,
                 lw_ref, lb_ref, g_ref):
    f = fac_ref[...]
    gate = jnp.dot(f, gw_ref[...], preferred_element_type=jnp.float32) + gb_ref[...]
    mask = batch_ref[...] == lax.broadcasted_iota(jnp.int32, (NF, G), 1)
    gmax = jnp.max(jnp.where(mask, gate, jnp.float32(-1e30)), axis=0, keepdims=True)
    gmax_f = jnp.sum(jnp.where(mask, gmax, 0.0), axis=1, keepdims=True)
    ex = jnp.exp(gate - gmax_f)
    denom = jnp.sum(jnp.where(mask, ex, 0.0), axis=0, keepdims=True)
    denom_f = jnp.sum(jnp.where(mask, denom, 0.0), axis=1, keepdims=True)
    alpha = ex / denom_f
    val = jnp.dot(f, nnw_ref[...], preferred_element_type=jnp.float32) + nnb_ref[...]
    wmat = jnp.where(mask, alpha, 0.0)
    g_agg = lax.dot_general(wmat, val, (((0,), (0,)), ((), ())),
                            preferred_element_type=jnp.float32)
    g = jnp.dot(g_agg, lw_ref[...], preferred_element_type=jnp.float32) + lb_ref[...]
    g_ref[...] = jnp.maximum(g, 0.0)


def _row_spec():
    return pl.BlockSpec((_BS, D), lambda i: (i, 0))


_W_SPEC = pl.BlockSpec((D, D), lambda i: (0, 0))
_B_SPEC = pl.BlockSpec((1, D), lambda i: (0, 0))
_AGG_SPEC = pl.BlockSpec((NCORE, _BS, D), lambda i: (0, i, 0))


def _pre_call(factors, variables, wia, ba, wjb, wic, bc, wjd):
    return pl.pallas_call(
        _pre_body,
        grid=(_GRID,),
        in_specs=[_row_spec(), _row_spec(), _W_SPEC, _B_SPEC, _W_SPEC,
                  _W_SPEC, _B_SPEC, _W_SPEC],
        out_specs=[_row_spec(), _row_spec(), _row_spec(), _row_spec()],
        out_shape=[jax.ShapeDtypeStruct((NF, D), jnp.float32),
                   jax.ShapeDtypeStruct((NV, D), jnp.float32),
                   jax.ShapeDtypeStruct((NV, D), jnp.float32),
                   jax.ShapeDtypeStruct((NF, D), jnp.float32)],
    )(factors, variables, wia, ba, wjb, wic, bc, wjd)


def _comb_call(factors, variables, aggfP, aggvP, wf1, wf2, bf, wv1, wv2, bv):
    return pl.pallas_call(
        _comb_body,
        grid=(_GRID,),
        in_specs=[_row_spec(), _row_spec(), _AGG_SPEC, _AGG_SPEC,
                  _W_SPEC, _W_SPEC, _B_SPEC, _W_SPEC, _W_SPEC, _B_SPEC],
        out_specs=[_row_spec(), _row_spec()],
        out_shape=[jax.ShapeDtypeStruct((NF, D), jnp.float32),
                   jax.ShapeDtypeStruct((NV, D), jnp.float32)],
    )(factors, variables, aggfP, aggvP, wf1, wf2, bf, wv1, wv2, bv)


def _global_call(factors, batch2d, gw, gb, nnw, nnb, lw, lb):
    return pl.pallas_call(
        _global_body,
        in_specs=[pl.BlockSpec((NF, D), lambda: (0, 0)),
                  pl.BlockSpec((NF, 1), lambda: (0, 0)),
                  pl.BlockSpec((D, 1), lambda: (0, 0)),
                  pl.BlockSpec((1, 1), lambda: (0, 0)),
                  pl.BlockSpec((D, D), lambda: (0, 0)),
                  pl.BlockSpec((1, D), lambda: (0, 0)),
                  pl.BlockSpec((D, D), lambda: (0, 0)),
                  pl.BlockSpec((1, D), lambda: (0, 0))],
        out_specs=pl.BlockSpec((G, D), lambda: (0, 0)),
        out_shape=jax.ShapeDtypeStruct((G, D), jnp.float32),
    )(factors, batch2d, gw, gb, nnw, nnb, lw, lb)


def kernel(variables, factors, edge_index, edge_attr, batch_idx,
           v2f_msg_W, v2f_msg_b, v2f_comb_W, v2f_comb_b,
           f2v_msg_W, f2v_msg_b, f2v_comb_W, f2v_comb_b,
           gate_W, gate_b, nn_W, nn_b, lin_W, lin_b):
    del edge_attr
    fac_idx = edge_index[1].astype(jnp.int32).reshape(NW, NCHUNK, K)
    var_idx = edge_index[0].astype(jnp.int32).reshape(NW, NCHUNK, K)
    batch2d = batch_idx.astype(jnp.int32).reshape(NF, 1)

    for l in range(2):
        a_tab, b_tab, c_tab, d_tab = _pre_call(
            factors, variables,
            v2f_msg_W[l, :D], v2f_msg_b[l].reshape(1, D), v2f_msg_W[l, D:],
            f2v_msg_W[l, :D], f2v_msg_b[l].reshape(1, D), f2v_msg_W[l, D:])
        aggf, aggv = _edges_call(a_tab, b_tab, c_tab, d_tab, fac_idx, var_idx)
        factors, variables = _comb_call(
            factors, variables, aggf, aggv,
            v2f_comb_W[l, :D], v2f_comb_W[l, D:], v2f_comb_b[l].reshape(1, D),
            f2v_comb_W[l, :D], f2v_comb_W[l, D:], f2v_comb_b[l].reshape(1, D))

    g = _global_call(factors, batch2d, gate_W, gate_b.reshape(1, 1),
                     nn_W, nn_b.reshape(1, D), lin_W[:D], lin_b.reshape(1, D))
    return (variables, factors, g)
